# baseline (speedup 1.0000x reference)
import functools

import jax
import jax.numpy as jnp
from jax import lax
from jax.experimental import pallas as pl
from jax.experimental.pallas import tpu as pltpu
from jax.experimental.pallas import tpu_sc as plsc

N = 10000
E = 320000
D = 128

NC = 2
NS = 16
NW = NC * NS
LANES = 16

CH = 112
NCHUNK = 90
TPE = NCHUNK * CH
EPAD = NW * TPE
NPAD = 10112
RPT = NPAD // NS
NPAD_DEG = 10240
RPT_DEG = NPAD_DEG // NS
BLK = 1000

_mesh = plsc.VectorSubcoreMesh(core_axis_name="c", subcore_axis_name="s")


@functools.partial(
    pl.kernel,
    mesh=_mesh,
    out_type=jax.ShapeDtypeStruct((NC, NPAD_DEG), jnp.float32),
    compiler_params=pltpu.CompilerParams(needs_layout_passes=False),
    scratch_types=[
        pltpu.VMEM_SHARED((NPAD_DEG,), jnp.float32),
        pltpu.VMEM((NCHUNK, CH), jnp.int32),
        pltpu.VMEM((NCHUNK, CH), jnp.float32),
        pltpu.VMEM((RPT_DEG,), jnp.float32),
        pltpu.SemaphoreType.DMA,
    ],
)
def _deg_kernel(col_hbm, ew_hbm, deg_out, dacc, cbuf, wbuf, zbuf, dsem):
    cid = lax.axis_index("c")
    sid = lax.axis_index("s")
    wid = cid * NS + sid
    pltpu.sync_copy(col_hbm.at[wid], cbuf)
    pltpu.sync_copy(ew_hbm.at[wid], wbuf)
    zeros16 = jnp.zeros((LANES,), jnp.float32)

    def _zero(i, carry):
        zbuf[pl.ds(i * LANES, LANES)] = zeros16
        return carry

    lax.fori_loop(0, RPT_DEG // LANES, _zero, 0)
    pltpu.sync_copy(zbuf, dacc.at[pl.ds(sid * RPT_DEG, RPT_DEG)])
    plsc.subcore_barrier()

    def _scat(j, carry):
        pltpu.async_copy(wbuf.at[j], dacc.at[cbuf.at[j]], dsem, add=True)
        return carry

    lax.fori_loop(0, NCHUNK, _scat, 0)

    def _drain(j, carry):
        pltpu.make_async_copy(wbuf.at[0], dacc.at[cbuf.at[0]], dsem).wait()
        return carry

    lax.fori_loop(0, NCHUNK, _drain, 0)
    plsc.subcore_barrier()
    pltpu.sync_copy(dacc.at[pl.ds(sid * RPT_DEG, RPT_DEG)],
                    deg_out.at[cid, pl.ds(sid * RPT_DEG, RPT_DEG)])


@functools.partial(
    pl.kernel,
    mesh=_mesh,
    out_type=jax.ShapeDtypeStruct((NC, NPAD, D), jnp.float32),
    compiler_params=pltpu.CompilerParams(needs_layout_passes=False),
    scratch_types=[
        pltpu.VMEM_SHARED((NPAD, D), jnp.float32),
        pltpu.VMEM((4, 3 * D), jnp.int32),
        pltpu.VMEM((CH, D), jnp.float32),
        pltpu.VMEM((CH, D), jnp.float32),
        pltpu.VMEM((CH, D), jnp.float32),
        pltpu.SemaphoreType.DMA,
        pltpu.SemaphoreType.DMA,
        pltpu.SemaphoreType.DMA,
        pltpu.SemaphoreType.DMA,
        pltpu.SemaphoreType.DMA,
        pltpu.SemaphoreType.DMA,
        pltpu.SemaphoreType.DMA,
    ],
)
def _agg_kernel(g_hbm, edge_hbm, agg_out, acc,
                ebuf, buf0, buf1, buf2,
                gsem0, gsem1, gsem2, ssem0, ssem1, ssem2, esem):
    cid = lax.axis_index("c")
    sid = lax.axis_index("s")
    wid = cid * NS + sid
    bufs = (buf0, buf1, buf2)
    gsems = (gsem0, gsem1, gsem2)
    ssems = (ssem0, ssem1, ssem2)

    zeros16 = jnp.zeros((LANES,), jnp.float32)

    def _zero(r, carry):
        for f in range(D // LANES):
            buf0[r, pl.ds(f * LANES, LANES)] = zeros16
        return carry

    lax.fori_loop(0, CH, _zero, 0)
    for t in range(RPT // CH):
        pltpu.sync_copy(buf0, acc.at[pl.ds(sid * RPT + t * CH, CH)])
    rem = RPT - (RPT // CH) * CH
    if rem:
        pltpu.sync_copy(buf0.at[pl.ds(0, rem)],
                        acc.at[pl.ds(sid * RPT + (RPT // CH) * CH, rem)])
    plsc.subcore_barrier()

    def _e_load(j):
        pltpu.async_copy(edge_hbm.at[wid, j], ebuf.at[j % 4], esem)

    def _e_drain():
        pltpu.make_async_copy(edge_hbm.at[wid, 0], ebuf.at[0], esem).wait()

    def _rows_of(j):
        return ebuf.at[j % 4, pl.ds(0, CH)]

    def _cols_of(j):
        return ebuf.at[j % 4, pl.ds(D, CH)]

    def _gather_start(j, s):
        pltpu.async_copy(g_hbm.at[_rows_of(j)], bufs[s], gsems[s])

    def _gather_wait(j, s):
        pltpu.make_async_copy(g_hbm.at[_rows_of(j)], bufs[s], gsems[s]).wait()

    def _scatter_start(j, s):
        pltpu.async_copy(bufs[s], acc.at[_cols_of(j)], ssems[s], add=True)

    def _scatter_wait(j, s):
        pltpu.make_async_copy(bufs[s], acc.at[_cols_of(j)], ssems[s]).wait()

    def _scale(j, s):
        buf = bufs[s]
        s4 = j % 4

        def _edge(e, carry):
            w16i = plsc.load_gather(
                ebuf, [jnp.full((LANES,), s4, jnp.int32),
                       jnp.full((LANES,), 2 * D + e, jnp.int32)])
            w16 = plsc.bitcast(w16i, jnp.float32)
            for f in range(D // LANES):
                sl = pl.ds(f * LANES, LANES)
                buf[e, sl] = buf[e, sl] * w16
            return carry

        lax.fori_loop(0, CH, _edge, 0)

    for q in range(3):
        _e_load(q)
    for q in range(2):
        _e_drain()
        _gather_start(q, q)

    def _triple(t, carry):
        for u in range(3):
            j = t * 3 + u
            s = u
            s1 = (u + 2) % 3
            _gather_wait(j, s)
            _scale(j, s)
            _scatter_start(j, s)

            @pl.when(j >= 1)
            def _():
                _scatter_wait(j - 1, s1)

            @pl.when(j <= NCHUNK - 4)
            def _():
                _e_load(j + 3)

            @pl.when(j <= NCHUNK - 3)
            def _():
                _e_drain()
                _gather_start(j + 2, s1)

        return carry

    lax.fori_loop(0, NCHUNK // 3, _triple, 0)
    _scatter_wait(NCHUNK - 1, (NCHUNK - 1) % 3)
    plsc.subcore_barrier()
    pltpu.sync_copy(acc.at[pl.ds(sid * RPT, RPT)],
                    agg_out.at[cid, pl.ds(sid * RPT, RPT)])


def _mm1_body(x_ref, w_ref, d0_ref, d1_ref, g_ref, dv_ref):
    deg = d0_ref[...] + d1_ref[...] + 1.0
    dv = jnp.where(deg > 0, lax.rsqrt(jnp.where(deg > 0, deg, 1.0)), 0.0)
    h = jnp.dot(x_ref[...], w_ref[...], preferred_element_type=jnp.float32)
    g_ref[...] = h * dv
    dv_ref[...] = dv


def _mm2_body(agg_ref, g_ref, dv_ref, b_ref, w_ref, g2_ref):
    dv = dv_ref[...]
    pre = dv * (agg_ref[0] + agg_ref[1] + g_ref[...]) + b_ref[...]
    h = jnp.maximum(pre, 0.0)
    g2_ref[...] = dv * jnp.dot(h, w_ref[...], preferred_element_type=jnp.float32)


def _fin_body(agg_ref, g_ref, dv_ref, b_ref, o_ref):
    dv = dv_ref[...]
    pre = dv * (agg_ref[0] + agg_ref[1] + g_ref[...]) + b_ref[...]
    o_ref[...] = jnp.maximum(pre, 0.0)


_row_spec = pl.BlockSpec((BLK, D), lambda i: (i, 0))
_col_spec = pl.BlockSpec((BLK, 1), lambda i: (i, 0))
_agg_spec = pl.BlockSpec((2, BLK, D), lambda i: (0, i, 0))
_w_spec = pl.BlockSpec((D, D), lambda i: (0, 0))
_b_spec = pl.BlockSpec((1, D), lambda i: (0, 0))
_GRID = (N // BLK,)


def _mm1(x, w1t, d0, d1):
    return pl.pallas_call(
        _mm1_body,
        grid=_GRID,
        in_specs=[_row_spec, _w_spec, _col_spec, _col_spec],
        out_specs=[_row_spec, _col_spec],
        out_shape=[jax.ShapeDtypeStruct((N, D), jnp.float32),
                   jax.ShapeDtypeStruct((N, 1), jnp.float32)],
    )(x, w1t, d0, d1)


def _mm2(agg, g, dv, b, w2t):
    return pl.pallas_call(
        _mm2_body,
        grid=_GRID,
        in_specs=[_agg_spec, _row_spec, _col_spec, _b_spec, _w_spec],
        out_specs=[_row_spec],
        out_shape=[jax.ShapeDtypeStruct((N, D), jnp.float32)],
    )(agg, g, dv, b, w2t)[0]


def _fin(agg, g, dv, b):
    return pl.pallas_call(
        _fin_body,
        grid=_GRID,
        in_specs=[_agg_spec, _row_spec, _col_spec, _b_spec],
        out_specs=[_row_spec],
        out_shape=[jax.ShapeDtypeStruct((N, D), jnp.float32)],
    )(agg, g, dv, b)[0]


def kernel(x, edge_index, edge_weights, W1, b1, W2, b2):
    f32, i32 = jnp.float32, jnp.int32
    row = edge_index[0]
    col = edge_index[1]
    pad = EPAD - E
    ar = jnp.arange(pad, dtype=i32)
    row_p = jnp.concatenate([row, ar % N])
    col_p = jnp.concatenate([col, N + ar % (NPAD - N)])
    ew_p = jnp.concatenate([edge_weights.astype(f32), jnp.zeros((pad,), f32)])
    colarr = col_p.reshape(NW, NCHUNK, CH)
    ewarr = ew_p.reshape(NW, NCHUNK, CH)
    zpad = jnp.zeros((NW, NCHUNK, D - CH), i32)
    earr = jnp.concatenate(
        [row_p.reshape(NW, NCHUNK, CH), zpad,
         colarr, zpad,
         lax.bitcast_convert_type(ewarr, i32), zpad], axis=2)
    w1t = W1.astype(f32).T
    w2t = W2.astype(f32).T
    b1r = b1.astype(f32).reshape(1, D)
    b2r = b2.astype(f32).reshape(1, D)

    deg_parts = _deg_kernel(colarr, ewarr)
    d0 = deg_parts[0, :N].reshape(N, 1)
    d1 = deg_parts[1, :N].reshape(N, 1)

    g1, dv = _mm1(x.astype(f32), w1t, d0, d1)
    agg1 = _agg_kernel(g1, earr)
    g2 = _mm2(agg1, g1, dv, b1r, w2t)
    agg2 = _agg_kernel(g2, earr)
    return _fin(agg2, g2, dv, b2r)


# speedup vs baseline: 1.0116x; 1.0116x over previous
import functools

import jax
import jax.numpy as jnp
from jax import lax
from jax.experimental import pallas as pl
from jax.experimental.pallas import tpu as pltpu
from jax.experimental.pallas import tpu_sc as plsc

N = 10000
E = 320000
D = 128

NC = 2
NS = 16
NW = NC * NS
LANES = 16

CH = 112
NCHUNK = 90
TPE = NCHUNK * CH
EPAD = NW * TPE
NPAD = 10112
RPT = NPAD // NS
NPAD_DEG = 10240
RPT_DEG = NPAD_DEG // NS
BLK = 1000

_mesh = plsc.VectorSubcoreMesh(core_axis_name="c", subcore_axis_name="s")


@functools.partial(
    pl.kernel,
    mesh=_mesh,
    out_type=jax.ShapeDtypeStruct((NC, NPAD_DEG), jnp.float32),
    compiler_params=pltpu.CompilerParams(needs_layout_passes=False),
    scratch_types=[
        pltpu.VMEM_SHARED((NPAD_DEG,), jnp.float32),
        pltpu.VMEM((NCHUNK, CH), jnp.int32),
        pltpu.VMEM((NCHUNK, CH), jnp.float32),
        pltpu.VMEM((RPT_DEG,), jnp.float32),
        pltpu.SemaphoreType.DMA,
    ],
)
def _deg_kernel(col_hbm, ew_hbm, deg_out, dacc, cbuf, wbuf, zbuf, dsem):
    cid = lax.axis_index("c")
    sid = lax.axis_index("s")
    wid = cid * NS + sid
    pltpu.sync_copy(col_hbm.at[wid], cbuf)
    pltpu.sync_copy(ew_hbm.at[wid], wbuf)
    zeros16 = jnp.zeros((LANES,), jnp.float32)

    def _zero(i, carry):
        zbuf[pl.ds(i * LANES, LANES)] = zeros16
        return carry

    lax.fori_loop(0, RPT_DEG // LANES, _zero, 0)
    pltpu.sync_copy(zbuf, dacc.at[pl.ds(sid * RPT_DEG, RPT_DEG)])
    plsc.subcore_barrier()

    def _scat(j, carry):
        pltpu.async_copy(wbuf.at[j], dacc.at[cbuf.at[j]], dsem, add=True)
        return carry

    lax.fori_loop(0, NCHUNK, _scat, 0)

    def _drain(j, carry):
        pltpu.make_async_copy(wbuf.at[0], dacc.at[cbuf.at[0]], dsem).wait()
        return carry

    lax.fori_loop(0, NCHUNK, _drain, 0)
    plsc.subcore_barrier()
    pltpu.sync_copy(dacc.at[pl.ds(sid * RPT_DEG, RPT_DEG)],
                    deg_out.at[cid, pl.ds(sid * RPT_DEG, RPT_DEG)])


@functools.partial(
    pl.kernel,
    mesh=_mesh,
    out_type=jax.ShapeDtypeStruct((NC, NPAD, D), jnp.float32),
    compiler_params=pltpu.CompilerParams(needs_layout_passes=False),
    scratch_types=[
        pltpu.VMEM_SHARED((NPAD, D), jnp.float32),
        pltpu.VMEM((3, CH), jnp.int32),
        pltpu.VMEM((3, CH), jnp.int32),
        pltpu.VMEM((3, CH), jnp.float32),
        pltpu.VMEM((CH, D), jnp.float32),
        pltpu.VMEM((CH, D), jnp.float32),
        pltpu.VMEM((CH, D), jnp.float32),
        pltpu.SemaphoreType.DMA,
        pltpu.SemaphoreType.DMA,
        pltpu.SemaphoreType.DMA,
        pltpu.SemaphoreType.DMA,
        pltpu.SemaphoreType.DMA,
        pltpu.SemaphoreType.DMA,
        pltpu.SemaphoreType.DMA,
        pltpu.SemaphoreType.DMA,
        pltpu.SemaphoreType.DMA,
        pltpu.SemaphoreType.DMA,
        pltpu.SemaphoreType.DMA,
        pltpu.SemaphoreType.DMA,
    ],
)
def _agg_kernel(g_hbm, row_hbm, col_hbm, ewb_hbm, agg_out, acc,
                rbuf, cbuf, wbuf, buf0, buf1, buf2,
                gsem0, gsem1, gsem2, ssem0, ssem1, ssem2,
                rsem0, rsem1, rsem2, csem0, csem1, csem2):
    cid = lax.axis_index("c")
    sid = lax.axis_index("s")
    wid = cid * NS + sid
    bufs = (buf0, buf1, buf2)
    gsems = (gsem0, gsem1, gsem2)
    ssems = (ssem0, ssem1, ssem2)
    rsems = (rsem0, rsem1, rsem2)
    csems = (csem0, csem1, csem2)

    zeros16 = jnp.zeros((LANES,), jnp.float32)

    def _zero(r, carry):
        for f in range(D // LANES):
            buf0[r, pl.ds(f * LANES, LANES)] = zeros16
        return carry

    lax.fori_loop(0, CH, _zero, 0)
    for t in range(RPT // CH):
        pltpu.sync_copy(buf0, acc.at[pl.ds(sid * RPT + t * CH, CH)])
    rem = RPT - (RPT // CH) * CH
    if rem:
        pltpu.sync_copy(buf0.at[pl.ds(0, rem)],
                        acc.at[pl.ds(sid * RPT + (RPT // CH) * CH, rem)])
    plsc.subcore_barrier()

    def _row_load(j, s):
        pltpu.async_copy(row_hbm.at[wid, j], rbuf.at[s], rsems[s])

    def _row_wait(j, s):
        pltpu.make_async_copy(row_hbm.at[wid, j], rbuf.at[s], rsems[s]).wait()

    def _ce_load(j, s):
        pltpu.async_copy(col_hbm.at[wid, j], cbuf.at[s], csems[s])
        pltpu.async_copy(ewb_hbm.at[wid, j], wbuf.at[s], csems[s])

    def _ce_wait(j, s):
        pltpu.make_async_copy(col_hbm.at[wid, j], cbuf.at[s], csems[s]).wait()
        pltpu.make_async_copy(ewb_hbm.at[wid, j], wbuf.at[s], csems[s]).wait()

    def _gather_start(s):
        pltpu.async_copy(g_hbm.at[rbuf.at[s]], bufs[s], gsems[s])

    def _gather_wait(s):
        pltpu.make_async_copy(g_hbm.at[rbuf.at[s]], bufs[s], gsems[s]).wait()

    def _scatter_start(s):
        pltpu.async_copy(bufs[s], acc.at[cbuf.at[s]], ssems[s], add=True)

    def _scatter_wait(s):
        pltpu.make_async_copy(bufs[s], acc.at[cbuf.at[s]], ssems[s]).wait()

    def _scale(s):
        buf = bufs[s]

        def _edge(e, carry):
            w16 = plsc.load_gather(
                wbuf, [jnp.full((LANES,), s, jnp.int32),
                       jnp.full((LANES,), e, jnp.int32)])
            for f in range(D // LANES):
                sl = pl.ds(f * LANES, LANES)
                buf[e, sl] = buf[e, sl] * w16
            return carry

        lax.fori_loop(0, CH, _edge, 0)

    for s in range(3):
        _row_load(s, s)
    for s in range(2):
        _ce_load(s, s)
    for s in range(2):
        _row_wait(s, s)
        _gather_start(s)

    def _triple(t, carry):
        for u in range(3):
            j = t * 3 + u
            s = u
            s1 = (u + 2) % 3
            _gather_wait(s)

            @pl.when(j <= NCHUNK - 4)
            def _():
                _row_load(j + 3, s)

            _ce_wait(j, s)
            _scale(s)
            _scatter_start(s)

            @pl.when(j >= 1)
            def _():
                _scatter_wait(s1)

            @pl.when(j <= NCHUNK - 3)
            def _():
                _ce_load(j + 2, s1)
                _row_wait(j + 2, s1)
                _gather_start(s1)

        return carry

    lax.fori_loop(0, NCHUNK // 3, _triple, 0)
    _scatter_wait((NCHUNK - 1) % 3)
    plsc.subcore_barrier()
    pltpu.sync_copy(acc.at[pl.ds(sid * RPT, RPT)],
                    agg_out.at[cid, pl.ds(sid * RPT, RPT)])


def _mm1_body(x_ref, w_ref, d0_ref, d1_ref, g_ref, dv_ref):
    deg = d0_ref[...] + d1_ref[...] + 1.0
    dv = jnp.where(deg > 0, lax.rsqrt(jnp.where(deg > 0, deg, 1.0)), 0.0)
    h = jnp.dot(x_ref[...], w_ref[...], preferred_element_type=jnp.float32)
    g_ref[...] = h * dv
    dv_ref[...] = dv


def _mm2_body(agg_ref, g_ref, dv_ref, b_ref, w_ref, g2_ref):
    dv = dv_ref[...]
    pre = dv * (agg_ref[0] + agg_ref[1] + g_ref[...]) + b_ref[...]
    h = jnp.maximum(pre, 0.0)
    g2_ref[...] = dv * jnp.dot(h, w_ref[...], preferred_element_type=jnp.float32)


def _fin_body(agg_ref, g_ref, dv_ref, b_ref, o_ref):
    dv = dv_ref[...]
    pre = dv * (agg_ref[0] + agg_ref[1] + g_ref[...]) + b_ref[...]
    o_ref[...] = jnp.maximum(pre, 0.0)


_row_spec = pl.BlockSpec((BLK, D), lambda i: (i, 0))
_col_spec = pl.BlockSpec((BLK, 1), lambda i: (i, 0))
_agg_spec = pl.BlockSpec((2, BLK, D), lambda i: (0, i, 0))
_w_spec = pl.BlockSpec((D, D), lambda i: (0, 0))
_b_spec = pl.BlockSpec((1, D), lambda i: (0, 0))
_GRID = (N // BLK,)


def _mm1(x, w1t, d0, d1):
    return pl.pallas_call(
        _mm1_body,
        grid=_GRID,
        in_specs=[_row_spec, _w_spec, _col_spec, _col_spec],
        out_specs=[_row_spec, _col_spec],
        out_shape=[jax.ShapeDtypeStruct((N, D), jnp.float32),
                   jax.ShapeDtypeStruct((N, 1), jnp.float32)],
    )(x, w1t, d0, d1)


def _mm2(agg, g, dv, b, w2t):
    return pl.pallas_call(
        _mm2_body,
        grid=_GRID,
        in_specs=[_agg_spec, _row_spec, _col_spec, _b_spec, _w_spec],
        out_specs=[_row_spec],
        out_shape=[jax.ShapeDtypeStruct((N, D), jnp.float32)],
    )(agg, g, dv, b, w2t)[0]


def _fin(agg, g, dv, b):
    return pl.pallas_call(
        _fin_body,
        grid=_GRID,
        in_specs=[_agg_spec, _row_spec, _col_spec, _b_spec],
        out_specs=[_row_spec],
        out_shape=[jax.ShapeDtypeStruct((N, D), jnp.float32)],
    )(agg, g, dv, b)[0]


def kernel(x, edge_index, edge_weights, W1, b1, W2, b2):
    f32, i32 = jnp.float32, jnp.int32
    row = edge_index[0]
    col = edge_index[1]
    pad = EPAD - E
    ar = jnp.arange(pad, dtype=i32)
    row_p = jnp.concatenate([row, ar % N])
    col_p = jnp.concatenate([col, N + ar % (NPAD - N)])
    ew_p = jnp.concatenate([edge_weights.astype(f32), jnp.zeros((pad,), f32)])
    rowarr = row_p.reshape(NW, NCHUNK, CH)
    colarr = col_p.reshape(NW, NCHUNK, CH)
    ewarr = ew_p.reshape(NW, NCHUNK, CH)
    w1t = W1.astype(f32).T
    w2t = W2.astype(f32).T
    b1r = b1.astype(f32).reshape(1, D)
    b2r = b2.astype(f32).reshape(1, D)

    deg_parts = _deg_kernel(colarr, ewarr)
    d0 = deg_parts[0, :N].reshape(N, 1)
    d1 = deg_parts[1, :N].reshape(N, 1)

    g1, dv = _mm1(x.astype(f32), w1t, d0, d1)
    agg1 = _agg_kernel(g1, rowarr, colarr, ewarr)
    g2 = _mm2(agg1, g1, dv, b1r, w2t)
    agg2 = _agg_kernel(g2, rowarr, colarr, ewarr)
    return _fin(agg2, g2, dv, b2r)


# speedup vs baseline: 1.0175x; 1.0058x over previous
import functools

import jax
import jax.numpy as jnp
from jax import lax
from jax.experimental import pallas as pl
from jax.experimental.pallas import tpu as pltpu
from jax.experimental.pallas import tpu_sc as plsc

N = 10000
E = 320000
D = 128

NC = 2
NS = 16
NW = NC * NS
LANES = 16

CH = 120
NCHUNK = 84
TPE = NCHUNK * CH
EPAD = NW * TPE
NPAD = 10112
RPT = NPAD // NS
NPAD_DEG = 10240
RPT_DEG = NPAD_DEG // NS
BLK = 1000

_mesh = plsc.VectorSubcoreMesh(core_axis_name="c", subcore_axis_name="s")


@functools.partial(
    pl.kernel,
    mesh=_mesh,
    out_type=jax.ShapeDtypeStruct((NC, NPAD_DEG), jnp.float32),
    compiler_params=pltpu.CompilerParams(needs_layout_passes=False),
    scratch_types=[
        pltpu.VMEM_SHARED((NPAD_DEG,), jnp.float32),
        pltpu.VMEM((NCHUNK, CH), jnp.int32),
        pltpu.VMEM((NCHUNK, CH), jnp.float32),
        pltpu.VMEM((RPT_DEG,), jnp.float32),
        pltpu.SemaphoreType.DMA,
    ],
)
def _deg_kernel(col_hbm, ew_hbm, deg_out, dacc, cbuf, wbuf, zbuf, dsem):
    cid = lax.axis_index("c")
    sid = lax.axis_index("s")
    wid = cid * NS + sid
    pltpu.sync_copy(col_hbm.at[wid], cbuf)
    pltpu.sync_copy(ew_hbm.at[wid], wbuf)
    zeros16 = jnp.zeros((LANES,), jnp.float32)

    def _zero(i, carry):
        zbuf[pl.ds(i * LANES, LANES)] = zeros16
        return carry

    lax.fori_loop(0, RPT_DEG // LANES, _zero, 0)
    pltpu.sync_copy(zbuf, dacc.at[pl.ds(sid * RPT_DEG, RPT_DEG)])
    plsc.subcore_barrier()

    def _scat(j, carry):
        pltpu.async_copy(wbuf.at[j], dacc.at[cbuf.at[j]], dsem, add=True)
        return carry

    lax.fori_loop(0, NCHUNK, _scat, 0)

    def _drain(j, carry):
        pltpu.make_async_copy(wbuf.at[0], dacc.at[cbuf.at[0]], dsem).wait()
        return carry

    lax.fori_loop(0, NCHUNK, _drain, 0)
    plsc.subcore_barrier()
    pltpu.sync_copy(dacc.at[pl.ds(sid * RPT_DEG, RPT_DEG)],
                    deg_out.at[cid, pl.ds(sid * RPT_DEG, RPT_DEG)])


@functools.partial(
    pl.kernel,
    mesh=_mesh,
    out_type=jax.ShapeDtypeStruct((NC, NPAD, D), jnp.float32),
    compiler_params=pltpu.CompilerParams(needs_layout_passes=False),
    scratch_types=[
        pltpu.VMEM_SHARED((NPAD, D), jnp.float32),
        pltpu.VMEM((3, CH), jnp.int32),
        pltpu.VMEM((3, CH), jnp.int32),
        pltpu.VMEM((3, CH), jnp.float32),
        pltpu.VMEM((CH, D), jnp.float32),
        pltpu.VMEM((CH, D), jnp.float32),
        pltpu.VMEM((CH, D), jnp.float32),
        pltpu.SemaphoreType.DMA,
        pltpu.SemaphoreType.DMA,
        pltpu.SemaphoreType.DMA,
        pltpu.SemaphoreType.DMA,
        pltpu.SemaphoreType.DMA,
        pltpu.SemaphoreType.DMA,
        pltpu.SemaphoreType.DMA,
        pltpu.SemaphoreType.DMA,
        pltpu.SemaphoreType.DMA,
        pltpu.SemaphoreType.DMA,
        pltpu.SemaphoreType.DMA,
        pltpu.SemaphoreType.DMA,
    ],
)
def _agg_kernel(g_hbm, row_hbm, col_hbm, ewb_hbm, agg_out, acc,
                rbuf, cbuf, wbuf, buf0, buf1, buf2,
                gsem0, gsem1, gsem2, ssem0, ssem1, ssem2,
                rsem0, rsem1, rsem2, csem0, csem1, csem2):
    cid = lax.axis_index("c")
    sid = lax.axis_index("s")
    wid = cid * NS + sid
    bufs = (buf0, buf1, buf2)
    gsems = (gsem0, gsem1, gsem2)
    ssems = (ssem0, ssem1, ssem2)
    rsems = (rsem0, rsem1, rsem2)
    csems = (csem0, csem1, csem2)

    zeros16 = jnp.zeros((LANES,), jnp.float32)

    def _zero(r, carry):
        for f in range(D // LANES):
            buf0[r, pl.ds(f * LANES, LANES)] = zeros16
        return carry

    lax.fori_loop(0, CH, _zero, 0)
    for t in range(RPT // CH):
        pltpu.sync_copy(buf0, acc.at[pl.ds(sid * RPT + t * CH, CH)])
    rem = RPT - (RPT // CH) * CH
    if rem:
        pltpu.sync_copy(buf0.at[pl.ds(0, rem)],
                        acc.at[pl.ds(sid * RPT + (RPT // CH) * CH, rem)])
    plsc.subcore_barrier()

    def _row_load(j, s):
        pltpu.async_copy(row_hbm.at[wid, j], rbuf.at[s], rsems[s])

    def _row_wait(j, s):
        pltpu.make_async_copy(row_hbm.at[wid, j], rbuf.at[s], rsems[s]).wait()

    def _ce_load(j, s):
        pltpu.async_copy(col_hbm.at[wid, j], cbuf.at[s], csems[s])
        pltpu.async_copy(ewb_hbm.at[wid, j], wbuf.at[s], csems[s])

    def _ce_wait(j, s):
        pltpu.make_async_copy(col_hbm.at[wid, j], cbuf.at[s], csems[s]).wait()
        pltpu.make_async_copy(ewb_hbm.at[wid, j], wbuf.at[s], csems[s]).wait()

    def _gather_start(s):
        pltpu.async_copy(g_hbm.at[rbuf.at[s]], bufs[s], gsems[s])

    def _gather_wait(s):
        pltpu.make_async_copy(g_hbm.at[rbuf.at[s]], bufs[s], gsems[s]).wait()

    def _scatter_start(s):
        pltpu.async_copy(bufs[s], acc.at[cbuf.at[s]], ssems[s], add=True)

    def _scatter_wait(s):
        pltpu.make_async_copy(bufs[s], acc.at[cbuf.at[s]], ssems[s]).wait()

    def _scale(s):
        buf = bufs[s]

        def _edge(e, carry):
            w16 = plsc.load_gather(
                wbuf, [jnp.full((LANES,), s, jnp.int32),
                       jnp.full((LANES,), e, jnp.int32)])
            for f in range(D // LANES):
                sl = pl.ds(f * LANES, LANES)
                buf[e, sl] = buf[e, sl] * w16
            return carry

        lax.fori_loop(0, CH, _edge, 0)

    for s in range(3):
        _row_load(s, s)
    for s in range(2):
        _ce_load(s, s)
    for s in range(2):
        _row_wait(s, s)
        _gather_start(s)

    def _triple(t, carry):
        for u in range(3):
            j = t * 3 + u
            s = u
            s1 = (u + 2) % 3
            _gather_wait(s)

            @pl.when(j <= NCHUNK - 4)
            def _():
                _row_load(j + 3, s)

            _ce_wait(j, s)
            _scale(s)
            _scatter_start(s)

            @pl.when(j >= 1)
            def _():
                _scatter_wait(s1)

            @pl.when(j <= NCHUNK - 3)
            def _():
                _ce_load(j + 2, s1)
                _row_wait(j + 2, s1)
                _gather_start(s1)

        return carry

    lax.fori_loop(0, NCHUNK // 3, _triple, 0)
    _scatter_wait((NCHUNK - 1) % 3)
    plsc.subcore_barrier()
    pltpu.sync_copy(acc.at[pl.ds(sid * RPT, RPT)],
                    agg_out.at[cid, pl.ds(sid * RPT, RPT)])


def _mm1_body(x_ref, w_ref, d0_ref, d1_ref, g_ref, dv_ref):
    deg = d0_ref[...] + d1_ref[...] + 1.0
    dv = jnp.where(deg > 0, lax.rsqrt(jnp.where(deg > 0, deg, 1.0)), 0.0)
    h = jnp.dot(x_ref[...], w_ref[...], preferred_element_type=jnp.float32)
    g_ref[...] = h * dv
    dv_ref[...] = dv


def _mm2_body(agg_ref, g_ref, dv_ref, b_ref, w_ref, g2_ref):
    dv = dv_ref[...]
    pre = dv * (agg_ref[0] + agg_ref[1] + g_ref[...]) + b_ref[...]
    h = jnp.maximum(pre, 0.0)
    g2_ref[...] = dv * jnp.dot(h, w_ref[...], preferred_element_type=jnp.float32)


def _fin_body(agg_ref, g_ref, dv_ref, b_ref, o_ref):
    dv = dv_ref[...]
    pre = dv * (agg_ref[0] + agg_ref[1] + g_ref[...]) + b_ref[...]
    o_ref[...] = jnp.maximum(pre, 0.0)


_row_spec = pl.BlockSpec((BLK, D), lambda i: (i, 0))
_col_spec = pl.BlockSpec((BLK, 1), lambda i: (i, 0))
_agg_spec = pl.BlockSpec((2, BLK, D), lambda i: (0, i, 0))
_w_spec = pl.BlockSpec((D, D), lambda i: (0, 0))
_b_spec = pl.BlockSpec((1, D), lambda i: (0, 0))
_GRID = (N // BLK,)


def _mm1(x, w1t, d0, d1):
    return pl.pallas_call(
        _mm1_body,
        grid=_GRID,
        in_specs=[_row_spec, _w_spec, _col_spec, _col_spec],
        out_specs=[_row_spec, _col_spec],
        out_shape=[jax.ShapeDtypeStruct((N, D), jnp.float32),
                   jax.ShapeDtypeStruct((N, 1), jnp.float32)],
    )(x, w1t, d0, d1)


def _mm2(agg, g, dv, b, w2t):
    return pl.pallas_call(
        _mm2_body,
        grid=_GRID,
        in_specs=[_agg_spec, _row_spec, _col_spec, _b_spec, _w_spec],
        out_specs=[_row_spec],
        out_shape=[jax.ShapeDtypeStruct((N, D), jnp.float32)],
    )(agg, g, dv, b, w2t)[0]


def _fin(agg, g, dv, b):
    return pl.pallas_call(
        _fin_body,
        grid=_GRID,
        in_specs=[_agg_spec, _row_spec, _col_spec, _b_spec],
        out_specs=[_row_spec],
        out_shape=[jax.ShapeDtypeStruct((N, D), jnp.float32)],
    )(agg, g, dv, b)[0]


def kernel(x, edge_index, edge_weights, W1, b1, W2, b2):
    f32, i32 = jnp.float32, jnp.int32
    row = edge_index[0]
    col = edge_index[1]
    pad = EPAD - E
    ar = jnp.arange(pad, dtype=i32)
    row_p = jnp.concatenate([row, ar % N])
    col_p = jnp.concatenate([col, N + ar % (NPAD - N)])
    ew_p = jnp.concatenate([edge_weights.astype(f32), jnp.zeros((pad,), f32)])
    rowarr = row_p.reshape(NW, NCHUNK, CH)
    colarr = col_p.reshape(NW, NCHUNK, CH)
    ewarr = ew_p.reshape(NW, NCHUNK, CH)
    w1t = W1.astype(f32).T
    w2t = W2.astype(f32).T
    b1r = b1.astype(f32).reshape(1, D)
    b2r = b2.astype(f32).reshape(1, D)

    deg_parts = _deg_kernel(colarr, ewarr)
    d0 = deg_parts[0, :N].reshape(N, 1)
    d1 = deg_parts[1, :N].reshape(N, 1)

    g1, dv = _mm1(x.astype(f32), w1t, d0, d1)
    agg1 = _agg_kernel(g1, rowarr, colarr, ewarr)
    g2 = _mm2(agg1, g1, dv, b1r, w2t)
    agg2 = _agg_kernel(g2, rowarr, colarr, ewarr)
    return _fin(agg2, g2, dv, b2r)
